# Initial kernel scaffold; baseline (speedup 1.0000x reference)
import jax
import jax.numpy as jnp
from jax.experimental import pallas as pl


def kernel(input_ids, emb, pos, Wi, bi, Wq, Wg, bg, Wc, bc, Wo, bo, cg, cb, Wk, bk, Wv, bv, Wwg, bwg, sem_k, sem_v, fg, fb, Wlm):
    raise NotImplementedError("write your pallas kernel here")


# speedup vs baseline: 2.2773x; 2.2773x over previous
import functools

import jax
import jax.numpy as jnp
from jax import lax
from jax.experimental import pallas as pl
from jax.experimental.pallas import tpu as pltpu
from jax.experimental.pallas import tpu_sc as plsc

_EPS_NORM = 1e-6
_EPS_LN = 1e-5
_WN = 16
_TK = 4


def _sc_gather(table, idx):
    n = idx.shape[0]
    d = table.shape[1]
    info = plsc.get_sparse_core_info()
    nw = info.num_cores * info.num_subcores
    b_per_w = n // nw
    mesh = plsc.VectorSubcoreMesh(core_axis_name="c", subcore_axis_name="s")

    @functools.partial(
        pl.kernel,
        mesh=mesh,
        compiler_params=pltpu.CompilerParams(use_tc_tiling_on_sc=False),
        out_type=jax.ShapeDtypeStruct((n, d), jnp.float32),
        scratch_types=[
            pltpu.VMEM((b_per_w,), jnp.int32),
            pltpu.VMEM((b_per_w, d), jnp.float32),
            pltpu.SemaphoreType.DMA,
        ],
    )
    def gk(table_hbm, idx_hbm, out_hbm, idx_v, rows_v, sem):
        wid = lax.axis_index("s") * info.num_cores + lax.axis_index("c")
        base = wid * b_per_w
        pltpu.sync_copy(idx_hbm.at[pl.ds(base, b_per_w)], idx_v)
        pltpu.async_copy(table_hbm.at[idx_v], rows_v, sem).wait()
        pltpu.sync_copy(rows_v, out_hbm.at[pl.ds(base, b_per_w)])

    return gk(table, idx)


def _bf(x):
    return x.astype(jnp.bfloat16)


def _bdot(a, b):
    return jnp.dot(_bf(a), _bf(b), preferred_element_type=jnp.float32)


def _ln2d(x, g, b):
    m = jnp.mean(x, axis=1, keepdims=True)
    d = x - m
    v = jnp.mean(d * d, axis=1, keepdims=True)
    return d / jnp.sqrt(v + _EPS_LN) * g + b


def _rec_kernel(nl, xg_ref, pos_ref, wi_ref, bi_ref, wq_ref, wg_ref, bg_ref,
                wc_ref, bc_ref, wo_ref, bo_ref, cg_ref, cb_ref,
                wkl_ref, bkl_ref, wvl_ref, bvl_ref, wwg_ref, bwg_ref,
                skt_ref, sv_ref, fg_ref, fb_ref,
                out_ref, x_scr, nskt_scr, mk_scr, mv_scr):
    s_len, bsz, dm = xg_ref.shape
    d = wi_ref.shape[1]
    sn = sv_ref.shape[1]
    ns = _WN + sn

    xfull = xg_ref[...] + pos_ref[...].reshape(s_len, 1, dm)
    x_all = _bdot(xfull.reshape(s_len * bsz, dm), wi_ref[...]) + bi_ref[...]
    x_scr[...] = x_all.reshape(s_len, bsz, d)

    for li in range(nl):
        kt = skt_ref[li]
        cn = jnp.sqrt(jnp.sum(kt * kt, axis=0, keepdims=True))
        nskt_scr[li] = kt / jnp.maximum(cn, _EPS_NORM)

    mk_scr[...] = jnp.zeros(mk_scr.shape, jnp.float32)
    mv_scr[...] = jnp.zeros(mv_scr.shape, jnp.float32)

    iota = lax.broadcasted_iota(jnp.int32, (bsz, ns), 1)
    neg_inf = jnp.float32(-jnp.inf)

    def step(t, hcarry):
        hcarry = list(hcarry)
        cur = x_scr[pl.ds(t, 1)][0]
        slot = lax.rem(t, _WN)
        for li in range(nl):
            prev = hcarry[li]
            base = li * _WN
            q0 = _bdot(jnp.concatenate([cur, prev], axis=1), wq_ref[li])
            qn = jnp.sqrt(jnp.sum(q0 * q0, axis=1, keepdims=True))
            q = q0 / jnp.maximum(qn, _EPS_NORM)
            qb = _bf(q).astype(jnp.float32)
            raws = []
            for k in range(_WN):
                kn = mk_scr[base + k]
                nrm = jnp.sqrt(jnp.sum(kn * kn, axis=1, keepdims=True))
                knn = _bf(kn / jnp.maximum(nrm, _EPS_NORM)).astype(jnp.float32)
                raws.append(jnp.sum(qb * knn, axis=1, keepdims=True))
            ws = jnp.concatenate(raws, axis=1)
            ss = jnp.dot(_bf(q), _bf(nskt_scr[li]),
                         preferred_element_type=jnp.float32)
            sc = jnp.concatenate([ws, ss], axis=1)
            sels = []
            tops = []
            for _ in range(_TK):
                mx = jnp.max(sc, axis=1, keepdims=True)
                cand = jnp.where(sc == mx, iota, ns)
                fi = jnp.min(cand, axis=1, keepdims=True)
                sel = iota == fi
                tops.append(mx)
                sels.append(sel)
                sc = jnp.where(sel, neg_inf, sc)
            es = [jnp.exp(tv - tops[0]) for tv in tops]
            z = es[0]
            for e in es[1:]:
                z = z + e
            wmap = jnp.zeros((bsz, ns), jnp.float32)
            for e, sel in zip(es, sels):
                wmap = wmap + jnp.where(sel, e / z, 0.0)
            mem = jnp.dot(wmap[:, _WN:], sv_ref[li],
                          preferred_element_type=jnp.float32,
                          precision=lax.Precision.HIGHEST)
            for k in range(_WN):
                mem = mem + wmap[:, k:k + 1] * mv_scr[base + k]
            joined = jnp.concatenate([cur, prev, mem], axis=1)
            gate = jax.nn.sigmoid(_bdot(joined, wg_ref[li]) + bg_ref[li])
            cnd = jnp.tanh(_bdot(joined, wc_ref[li]) + bc_ref[li])
            h = (1.0 - gate) * prev + gate * cnd
            proj = _bdot(jnp.concatenate([cur, h], axis=1), wo_ref[li]) + bo_ref[li]
            newh = _ln2d(proj + cur, cg_ref[li], cb_ref[li])
            nhb = _bf(newh).astype(jnp.float32)
            wwb = _bf(wwg_ref[li]).astype(jnp.float32)
            g2 = jax.nn.sigmoid(
                jnp.sum(nhb * wwb, axis=1, keepdims=True) + bwg_ref[li])
            nk = jnp.tanh(_bdot(newh, wkl_ref[li]) + bkl_ref[li])
            nv = jnp.tanh(_bdot(newh, wvl_ref[li]) + bvl_ref[li])
            ssl = base + slot
            ck = mk_scr[pl.ds(ssl, 1)][0]
            cv = mv_scr[pl.ds(ssl, 1)][0]
            mk_scr[pl.ds(ssl, 1)] = (ck * (1.0 - g2) + nk * g2)[None]
            mv_scr[pl.ds(ssl, 1)] = (cv * (1.0 - g2) + nv * g2)[None]
            hcarry[li] = newh
            cur = newh
        out_ref[pl.ds(t, 1)] = _ln2d(cur, fg_ref[...], fb_ref[...])[None]
        return tuple(hcarry)

    h0 = tuple(jnp.zeros((bsz, d), jnp.float32) for _ in range(nl))
    lax.fori_loop(0, s_len, step, h0)


def _logits_kernel(x_ref, w_ref, o_ref):
    o_ref[0] = jnp.dot(x_ref[0], w_ref[...], preferred_element_type=jnp.float32)


def _logits(h3, wlm, vt=2048):
    b, s, d = h3.shape
    v = wlm.shape[1]
    vt = min(vt, v)
    nv = (v + vt - 1) // vt
    return pl.pallas_call(
        _logits_kernel,
        grid=(nv, b),
        in_specs=[
            pl.BlockSpec((1, s, d), lambda j, i: (i, 0, 0)),
            pl.BlockSpec((d, vt), lambda j, i: (0, j)),
        ],
        out_specs=pl.BlockSpec((1, s, vt), lambda j, i: (i, 0, j)),
        out_shape=jax.ShapeDtypeStruct((b, s, v), jnp.float32),
    )(h3, wlm)


def kernel(input_ids, emb, pos, Wi, bi, Wq, Wg, bg, Wc, bc, Wo, bo, cg, cb,
           Wk, bk, Wv, bv, Wwg, bwg, sem_k, sem_v, fg, fb, Wlm):
    bsz, s_len = input_ids.shape
    dm = emb.shape[1]
    d = Wi.shape[1]
    nl = Wq.shape[0]
    sn = sem_k.shape[1]

    ids = input_ids.astype(jnp.int32).T.reshape(-1)
    xg = _sc_gather(emb, ids)
    xg3 = xg.reshape(s_len, bsz, dm)

    hs = pl.pallas_call(
        functools.partial(_rec_kernel, nl),
        out_shape=jax.ShapeDtypeStruct((s_len, bsz, d), jnp.float32),
        scratch_shapes=[
            pltpu.VMEM((s_len, bsz, d), jnp.float32),
            pltpu.VMEM((nl, d, sn), jnp.float32),
            pltpu.VMEM((nl * _WN, bsz, d), jnp.float32),
            pltpu.VMEM((nl * _WN, bsz, d), jnp.float32),
        ],
    )(
        xg3,
        pos[:s_len],
        Wi,
        bi.reshape(1, d),
        Wq,
        Wg,
        bg.reshape(nl, 1, d),
        Wc,
        bc.reshape(nl, 1, d),
        Wo,
        bo.reshape(nl, 1, d),
        cg.reshape(nl, 1, d),
        cb.reshape(nl, 1, d),
        Wk,
        bk.reshape(nl, 1, d),
        Wv,
        bv.reshape(nl, 1, d),
        Wwg.transpose(0, 2, 1),
        bwg.reshape(nl, 1, 1),
        sem_k.transpose(0, 2, 1),
        sem_v,
        fg.reshape(1, d),
        fb.reshape(1, d),
    )
    h3 = hs.transpose(1, 0, 2)
    return _logits(h3, Wlm)


# speedup vs baseline: 2.3084x; 1.0136x over previous
import functools

import jax
import jax.numpy as jnp
from jax import lax
from jax.experimental import pallas as pl
from jax.experimental.pallas import tpu as pltpu
from jax.experimental.pallas import tpu_sc as plsc

_EPS_NORM = 1e-6
_EPS_LN = 1e-5
_WN = 16
_TK = 4


def _sc_gather(table, idx):
    n = idx.shape[0]
    d = table.shape[1]
    info = plsc.get_sparse_core_info()
    nw = info.num_cores * info.num_subcores
    b_per_w = n // nw
    mesh = plsc.VectorSubcoreMesh(core_axis_name="c", subcore_axis_name="s")

    @functools.partial(
        pl.kernel,
        mesh=mesh,
        out_type=jax.ShapeDtypeStruct((n, d), jnp.float32),
        scratch_types=[
            pltpu.VMEM((b_per_w,), jnp.int32),
            pltpu.VMEM((b_per_w, d), jnp.float32),
            pltpu.SemaphoreType.DMA,
        ],
    )
    def gk(table_hbm, idx_hbm, out_hbm, idx_v, rows_v, sem):
        wid = lax.axis_index("s") * info.num_cores + lax.axis_index("c")
        base = wid * b_per_w
        pltpu.sync_copy(idx_hbm.at[pl.ds(base, b_per_w)], idx_v)
        pltpu.async_copy(table_hbm.at[idx_v], rows_v, sem).wait()
        pltpu.sync_copy(rows_v, out_hbm.at[pl.ds(base, b_per_w)])

    return gk(table, idx)


def _bf(x):
    return x.astype(jnp.bfloat16)


def _bdot(a, b):
    return jnp.dot(_bf(a), _bf(b), preferred_element_type=jnp.float32)


def _ln2d(x, g, b):
    m = jnp.mean(x, axis=1, keepdims=True)
    d = x - m
    v = jnp.mean(d * d, axis=1, keepdims=True)
    return d / jnp.sqrt(v + _EPS_LN) * g + b


def _rec_kernel(nl, xg_ref, pos_ref, wi_ref, bi_ref, wq_ref, wg_ref, bg_ref,
                wc_ref, bc_ref, wo_ref, bo_ref, cg_ref, cb_ref,
                wkl_ref, bkl_ref, wvl_ref, bvl_ref, wwg_ref, bwg_ref,
                skt_ref, sv_ref, fg_ref, fb_ref,
                out_ref, x_scr, nskt_scr, mk_scr, mv_scr):
    s_len, bsz, dm = xg_ref.shape
    d = wi_ref.shape[1]
    sn = sv_ref.shape[1]
    ns = _WN + sn

    xfull = xg_ref[...] + pos_ref[...].reshape(s_len, 1, dm)
    x_all = _bdot(xfull.reshape(s_len * bsz, dm), wi_ref[...]) + bi_ref[...]
    x_scr[...] = x_all.reshape(s_len, bsz, d)

    for li in range(nl):
        kt = skt_ref[li]
        cn = jnp.sqrt(jnp.sum(kt * kt, axis=0, keepdims=True))
        nskt_scr[li] = kt / jnp.maximum(cn, _EPS_NORM)

    mk_scr[...] = jnp.zeros(mk_scr.shape, jnp.float32)
    mv_scr[...] = jnp.zeros(mv_scr.shape, jnp.float32)

    iota = lax.broadcasted_iota(jnp.int32, (bsz, ns), 1)
    neg_inf = jnp.float32(-jnp.inf)

    def step(t, hcarry):
        hcarry = list(hcarry)
        cur = x_scr[pl.ds(t, 1)][0]
        slot = lax.rem(t, _WN)
        for li in range(nl):
            prev = hcarry[li]
            base = li * _WN
            q0 = _bdot(jnp.concatenate([cur, prev], axis=1), wq_ref[li])
            qn = jnp.sqrt(jnp.sum(q0 * q0, axis=1, keepdims=True))
            q = q0 / jnp.maximum(qn, _EPS_NORM)
            qb = _bf(q).astype(jnp.float32)
            raws = []
            for k in range(_WN):
                kn = mk_scr[base + k]
                nrm = jnp.sqrt(jnp.sum(kn * kn, axis=1, keepdims=True))
                knn = _bf(kn / jnp.maximum(nrm, _EPS_NORM)).astype(jnp.float32)
                raws.append(jnp.sum(qb * knn, axis=1, keepdims=True))
            ws = jnp.concatenate(raws, axis=1)
            ss = jnp.dot(_bf(q), _bf(nskt_scr[li]),
                         preferred_element_type=jnp.float32)
            sc = jnp.concatenate([ws, ss], axis=1)
            sels = []
            tops = []
            for _ in range(_TK):
                mx = jnp.max(sc, axis=1, keepdims=True)
                cand = jnp.where(sc == mx, iota, ns)
                fi = jnp.min(cand, axis=1, keepdims=True)
                sel = iota == fi
                tops.append(mx)
                sels.append(sel)
                sc = jnp.where(sel, neg_inf, sc)
            es = [jnp.exp(tv - tops[0]) for tv in tops]
            z = es[0]
            for e in es[1:]:
                z = z + e
            wmap = jnp.zeros((bsz, ns), jnp.float32)
            for e, sel in zip(es, sels):
                wmap = wmap + jnp.where(sel, e / z, 0.0)
            mem = jnp.dot(wmap[:, _WN:], sv_ref[li],
                          preferred_element_type=jnp.float32,
                          precision=lax.Precision.HIGHEST)
            for k in range(_WN):
                mem = mem + wmap[:, k:k + 1] * mv_scr[base + k]
            joined = jnp.concatenate([cur, prev, mem], axis=1)
            gate = jax.nn.sigmoid(_bdot(joined, wg_ref[li]) + bg_ref[li])
            cnd = jnp.tanh(_bdot(joined, wc_ref[li]) + bc_ref[li])
            h = (1.0 - gate) * prev + gate * cnd
            proj = _bdot(jnp.concatenate([cur, h], axis=1), wo_ref[li]) + bo_ref[li]
            newh = _ln2d(proj + cur, cg_ref[li], cb_ref[li])
            nhb = _bf(newh).astype(jnp.float32)
            wwb = _bf(wwg_ref[li]).astype(jnp.float32)
            g2 = jax.nn.sigmoid(
                jnp.sum(nhb * wwb, axis=1, keepdims=True) + bwg_ref[li])
            nk = jnp.tanh(_bdot(newh, wkl_ref[li]) + bkl_ref[li])
            nv = jnp.tanh(_bdot(newh, wvl_ref[li]) + bvl_ref[li])
            ssl = base + slot
            ck = mk_scr[pl.ds(ssl, 1)][0]
            cv = mv_scr[pl.ds(ssl, 1)][0]
            mk_scr[pl.ds(ssl, 1)] = (ck * (1.0 - g2) + nk * g2)[None]
            mv_scr[pl.ds(ssl, 1)] = (cv * (1.0 - g2) + nv * g2)[None]
            hcarry[li] = newh
            cur = newh
        out_ref[pl.ds(t, 1)] = _ln2d(cur, fg_ref[...], fb_ref[...])[None]
        return tuple(hcarry)

    h0 = tuple(jnp.zeros((bsz, d), jnp.float32) for _ in range(nl))
    lax.fori_loop(0, s_len, step, h0)


def _logits_kernel(x_ref, w_ref, o_ref):
    o_ref[0] = jnp.dot(x_ref[0], w_ref[...], preferred_element_type=jnp.float32)


def _logits(h3, wlm, vt=2048):
    b, s, d = h3.shape
    v = wlm.shape[1]
    vt = min(vt, v)
    nv = (v + vt - 1) // vt
    return pl.pallas_call(
        _logits_kernel,
        grid=(nv, b),
        in_specs=[
            pl.BlockSpec((1, s, d), lambda j, i: (i, 0, 0)),
            pl.BlockSpec((d, vt), lambda j, i: (0, j)),
        ],
        out_specs=pl.BlockSpec((1, s, vt), lambda j, i: (i, 0, j)),
        out_shape=jax.ShapeDtypeStruct((b, s, v), jnp.float32),
    )(h3, wlm)


def kernel(input_ids, emb, pos, Wi, bi, Wq, Wg, bg, Wc, bc, Wo, bo, cg, cb,
           Wk, bk, Wv, bv, Wwg, bwg, sem_k, sem_v, fg, fb, Wlm):
    bsz, s_len = input_ids.shape
    dm = emb.shape[1]
    d = Wi.shape[1]
    nl = Wq.shape[0]
    sn = sem_k.shape[1]

    ids = input_ids.astype(jnp.int32).T.reshape(-1)
    emb_p = jnp.pad(emb, ((0, 0), (0, 128 - dm)))
    xg = _sc_gather(emb_p, ids)[:, :dm]
    xg3 = xg.reshape(s_len, bsz, dm)

    hs = pl.pallas_call(
        functools.partial(_rec_kernel, nl),
        out_shape=jax.ShapeDtypeStruct((s_len, bsz, d), jnp.float32),
        scratch_shapes=[
            pltpu.VMEM((s_len, bsz, d), jnp.float32),
            pltpu.VMEM((nl, d, sn), jnp.float32),
            pltpu.VMEM((nl * _WN, bsz, d), jnp.float32),
            pltpu.VMEM((nl * _WN, bsz, d), jnp.float32),
        ],
    )(
        xg3,
        pos[:s_len],
        Wi,
        bi.reshape(1, d),
        Wq,
        Wg,
        bg.reshape(nl, 1, d),
        Wc,
        bc.reshape(nl, 1, d),
        Wo,
        bo.reshape(nl, 1, d),
        cg.reshape(nl, 1, d),
        cb.reshape(nl, 1, d),
        Wk,
        bk.reshape(nl, 1, d),
        Wv,
        bv.reshape(nl, 1, d),
        Wwg.transpose(0, 2, 1),
        bwg.reshape(nl, 1, 1),
        sem_k.transpose(0, 2, 1),
        sem_v,
        fg.reshape(1, d),
        fb.reshape(1, d),
    )
    h3 = hs.transpose(1, 0, 2)
    return _logits(h3, Wlm)
